# Initial kernel scaffold; baseline (speedup 1.0000x reference)
import jax
import jax.numpy as jnp
from jax.experimental import pallas as pl


def kernel(x, edge_index, neighbor_features, prev_time_features, W1, b1, W2, b2, Wn, Wt):
    raise NotImplementedError("write your pallas kernel here")


# speedup vs baseline: 9.8702x; 9.8702x over previous
import functools

import jax
import jax.numpy as jnp
from jax import lax
from jax.experimental import pallas as pl
from jax.experimental.pallas import tpu as pltpu
from jax.experimental.pallas import tpu_sc as plsc

N_NODES = 10000
DIM = 128
NC = 2
NS = 16
NW = NC * NS
CHUNK = 128
N_PAD = 10240
ROWS_PER_TILE = N_PAD // NS
CNT_W = 16
BLK = 128

def _mesh():
    return plsc.VectorSubcoreMesh(
        core_axis_name="c", subcore_axis_name="s", num_cores=NC, num_subcores=NS)


def _deg_body(nchunk, dst_hbm, zeros_hbm, out_hbm, idx_v, cnt_local):
    c = lax.axis_index("c")
    s = lax.axis_index("s")
    wid = c * NS + s
    pltpu.sync_copy(zeros_hbm, cnt_local)
    ones = jnp.ones((16,), jnp.float32)

    def body(j, carry):
        pltpu.sync_copy(dst_hbm.at[wid * nchunk + j], idx_v)
        for g in range(CHUNK // 16):
            iv = idx_v[pl.ds(g * 16, 16)]
            plsc.addupdate_scatter(cnt_local, [iv], ones)
        return carry

    lax.fori_loop(0, nchunk, body, 0)
    pltpu.sync_copy(cnt_local, out_hbm.at[wid])


def _gather_body(nchunk, src_hbm, dst_hbm, g_hbm, zeros_hbm, out_hbm,
                 sidx_v, didx_v, rows_v, acc_sh, sem):
    c = lax.axis_index("c")
    s = lax.axis_index("s")
    wid = c * NS + s
    pltpu.sync_copy(zeros_hbm, acc_sh.at[pl.ds(s * ROWS_PER_TILE, ROWS_PER_TILE)])
    plsc.subcore_barrier()

    def body(j, carry):
        row = wid * nchunk + j
        pltpu.sync_copy(src_hbm.at[row], sidx_v)
        pltpu.sync_copy(dst_hbm.at[row], didx_v)
        pltpu.async_copy(g_hbm.at[sidx_v], rows_v, sem).wait()
        pltpu.sync_copy(rows_v, acc_sh.at[didx_v], add=True)
        return carry

    lax.fori_loop(0, nchunk, body, 0)
    plsc.subcore_barrier()
    pltpu.sync_copy(acc_sh.at[pl.ds(s * ROWS_PER_TILE, ROWS_PER_TILE)],
                    out_hbm.at[pl.ds(c * N_PAD + s * ROWS_PER_TILE, ROWS_PER_TILE)])


def _deg_call(dst2, zeros_deg, nchunk):
    k = pl.kernel(
        functools.partial(_deg_body, nchunk),
        out_type=jax.ShapeDtypeStruct((NW, N_PAD), jnp.float32),
        mesh=_mesh(),
        scratch_types=[
            pltpu.VMEM((CHUNK,), jnp.int32),
            pltpu.VMEM((N_PAD,), jnp.float32),
        ],
        compiler_params=pltpu.CompilerParams(needs_layout_passes=False),
    )
    return k(dst2, zeros_deg)


def _gather_call(src2, dst2, g, zeros_rows, nchunk):
    k = pl.kernel(
        functools.partial(_gather_body, nchunk),
        out_type=jax.ShapeDtypeStruct((NC * N_PAD, DIM), jnp.float32),
        mesh=_mesh(),
        scratch_types=[
            pltpu.VMEM((CHUNK,), jnp.int32),
            pltpu.VMEM((CHUNK,), jnp.int32),
            pltpu.VMEM((CHUNK, DIM), jnp.float32),
            pltpu.VMEM_SHARED((N_PAD, DIM), jnp.float32),
            pltpu.SemaphoreType.DMA,
        ],
    )
    return k(src2, dst2, g, zeros_rows)


def _dinv(cnt):
    deg = jnp.sum(cnt, axis=0) + 1.0
    return lax.rsqrt(deg)[:, None]


def _tc1_body(x_ref, w1_ref, cnt_ref, g_ref):
    dinv = _dinv(cnt_ref[...])
    g_ref[...] = jnp.dot(x_ref[...], w1_ref[...],
                         preferred_element_type=jnp.float32) * dinv


def _tc2_body(acc_ref, g1_ref, cnt_ref, w2_ref, b1_ref, g2_ref):
    dinv = _dinv(cnt_ref[...])
    a = acc_ref[0] + acc_ref[1] + g1_ref[...]
    h = jnp.maximum(a * dinv + b1_ref[...], 0.0)
    g2_ref[...] = jnp.dot(h, w2_ref[...],
                          preferred_element_type=jnp.float32) * dinv


def _tc3_body(acc_ref, g2_ref, cnt_ref, b2_ref, nb_ref, pv_ref, wn_ref, wt_ref,
              out_ref):
    dinv = _dinv(cnt_ref[...])
    a = acc_ref[0] + acc_ref[1] + g2_ref[...]
    out_ref[...] = (a * dinv + b2_ref[...]
                    + jnp.dot(nb_ref[...], wn_ref[...],
                              preferred_element_type=jnp.float32)
                    + jnp.dot(pv_ref[...], wt_ref[...],
                              preferred_element_type=jnp.float32))


_ROW = pl.BlockSpec((BLK, DIM), lambda i: (i, 0))
_MAT = pl.BlockSpec((DIM, DIM), lambda i: (0, 0))
_CNT = pl.BlockSpec((NW, BLK), lambda i: (0, i))
_ACC = pl.BlockSpec((2, BLK, DIM), lambda i: (0, i, 0))
_BIAS = pl.BlockSpec((1, DIM), lambda i: (0, 0))
_GRID = (N_PAD // BLK,)
_OUT_ROWS = jax.ShapeDtypeStruct((N_PAD, DIM), jnp.float32)


def _tc1(x_pad, W1, counts):
    return pl.pallas_call(
        _tc1_body, grid=_GRID,
        in_specs=[_ROW, _MAT, _CNT], out_specs=_ROW,
        out_shape=_OUT_ROWS)(x_pad, W1, counts)


def _tc2(acc1, g1, counts, W2, b1):
    return pl.pallas_call(
        _tc2_body, grid=_GRID,
        in_specs=[_ACC, _ROW, _CNT, _MAT, _BIAS], out_specs=_ROW,
        out_shape=_OUT_ROWS)(acc1, g1, counts, W2, b1)


def _tc3(acc2, g2, counts, b2, nb, pv, Wn, Wt):
    return pl.pallas_call(
        _tc3_body, grid=_GRID,
        in_specs=[_ACC, _ROW, _CNT, _BIAS, _ROW, _ROW, _MAT, _MAT],
        out_specs=_ROW,
        out_shape=_OUT_ROWS)(acc2, g2, counts, b2, nb, pv, Wn, Wt)


def kernel(x, edge_index, neighbor_features, prev_time_features,
           W1, b1, W2, b2, Wn, Wt):
    E = edge_index.shape[1]
    nchunk = -(-E // (NW * CHUNK))
    e_pad = nchunk * NW * CHUNK
    ei = edge_index.astype(jnp.int32)
    pad = jnp.full((e_pad - E,), N_NODES, jnp.int32)
    src2 = jnp.concatenate([ei[0], pad]).reshape(nchunk * NW, CHUNK)
    dst2 = jnp.concatenate([ei[1], pad]).reshape(nchunk * NW, CHUNK)

    rpad = ((0, N_PAD - N_NODES), (0, 0))
    x_pad = jnp.pad(x, rpad)
    nb_pad = jnp.pad(neighbor_features, rpad)
    pv_pad = jnp.pad(prev_time_features, rpad)
    zeros_deg = jnp.zeros((N_PAD,), jnp.float32)
    zeros_rows = jnp.zeros((ROWS_PER_TILE, DIM), jnp.float32)

    counts = _deg_call(dst2, zeros_deg, nchunk)
    g1 = _tc1(x_pad, W1, counts)
    acc1 = _gather_call(src2, dst2, g1, zeros_rows, nchunk).reshape(NC, N_PAD, DIM)
    g2 = _tc2(acc1, g1, counts, W2, b1.reshape(1, DIM))
    acc2 = _gather_call(src2, dst2, g2, zeros_rows, nchunk).reshape(NC, N_PAD, DIM)
    out = _tc3(acc2, g2, counts, b2.reshape(1, DIM), nb_pad, pv_pad, Wn, Wt)
    return out[:N_NODES]
